# Initial kernel scaffold; baseline (speedup 1.0000x reference)
import functools

import jax
import jax.numpy as jnp
from jax import lax
from jax.experimental import pallas as pl
from jax.experimental.pallas import tpu as pltpu
from jax.experimental.pallas import tpu_sc as plsc

_N_USER = 10000
_N_ITEM = 10000
_D = 128
_E = 320000

_NUM_TILES = 16
_EDGES_PER_TILE = _E // _NUM_TILES
_CHUNK = 80
_NCHUNKS = _EDGES_PER_TILE // _CHUNK
_ROWS_PER_TILE = _N_USER // _NUM_TILES


def _sc_aggregate(table, src_all, dst_all, zeros):
    n = zeros.shape[0]
    mesh = plsc.VectorSubcoreMesh(core_axis_name="c", subcore_axis_name="s")

    @functools.partial(
        pl.kernel,
        out_type=jax.ShapeDtypeStruct((2, n, _D), jnp.float32),
        mesh=mesh,
        scratch_types=[
            pltpu.VMEM((_CHUNK,), jnp.int32),
            pltpu.VMEM((_CHUNK,), jnp.int32),
            pltpu.VMEM((_CHUNK, _D), jnp.float32),
            pltpu.VMEM_SHARED((n, _D), jnp.float32),
            pltpu.SemaphoreType.DMA,
        ],
    )
    def agg_kernel(table_hbm, src_hbm, dst_hbm, zeros_hbm, out_hbm,
                   src_v, dst_v, rows_v, acc_sh, sem):
        c = lax.axis_index("c")
        s = lax.axis_index("s")
        rbase = s * _ROWS_PER_TILE
        pltpu.sync_copy(zeros_hbm.at[pl.ds(rbase, _ROWS_PER_TILE)],
                        acc_sh.at[pl.ds(rbase, _ROWS_PER_TILE)])
        plsc.subcore_barrier()

        ebase = s * _EDGES_PER_TILE

        @pl.loop(0, _NCHUNKS)
        def _(i):
            e0 = ebase + i * _CHUNK
            pltpu.sync_copy(src_hbm.at[c, pl.ds(e0, _CHUNK)], src_v)
            pltpu.sync_copy(dst_hbm.at[c, pl.ds(e0, _CHUNK)], dst_v)
            pltpu.async_copy(table_hbm.at[src_v], rows_v, sem).wait()
            pltpu.sync_copy(rows_v, acc_sh.at[dst_v], add=True)

        plsc.subcore_barrier()
        pltpu.sync_copy(acc_sh.at[pl.ds(rbase, _ROWS_PER_TILE)],
                        out_hbm.at[c, pl.ds(rbase, _ROWS_PER_TILE)])

    return agg_kernel(table, src_all, dst_all, zeros)


def _affine_kernel(agg0_ref, agg1_ref, xi_ref, xu_ref,
                   wm0_ref, wr0_ref, b0_ref, wm1_ref, wr1_ref, b1_ref,
                   oi_ref, ou_ref):
    oi_ref[...] = (
        jnp.dot(agg0_ref[0], wm0_ref[...], preferred_element_type=jnp.float32)
        + jnp.dot(xi_ref[...], wr0_ref[...], preferred_element_type=jnp.float32)
        + b0_ref[...]
    )
    ou_ref[...] = (
        jnp.dot(agg1_ref[0], wm1_ref[...], preferred_element_type=jnp.float32)
        + jnp.dot(xu_ref[...], wr1_ref[...], preferred_element_type=jnp.float32)
        + b1_ref[...]
    )


def _tc_epilogue(agg, x_item, x_user, wm0, wr0, b0, wm1, wr1, b1):
    n = x_item.shape[0]
    blk = 2000
    grid = (n // blk,)
    row_spec = pl.BlockSpec((blk, _D), lambda i: (i, 0))
    w_spec = pl.BlockSpec((_D, _D), lambda i: (0, 0))
    b_spec = pl.BlockSpec((1, _D), lambda i: (0, 0))
    return pl.pallas_call(
        _affine_kernel,
        grid=grid,
        in_specs=[
            pl.BlockSpec((1, blk, _D), lambda i: (0, i, 0)),
            pl.BlockSpec((1, blk, _D), lambda i: (1, i, 0)),
            row_spec, row_spec,
            w_spec, w_spec, b_spec,
            w_spec, w_spec, b_spec,
        ],
        out_specs=[row_spec, row_spec],
        out_shape=[
            jax.ShapeDtypeStruct((n, _D), jnp.float32),
            jax.ShapeDtypeStruct((n, _D), jnp.float32),
        ],
    )(agg, agg, x_item, x_user, wm0, wr0, b0.reshape(1, _D),
      wm1, wr1, b1.reshape(1, _D))


def kernel(x_user, x_item, edge_index_u2i, edge_index_i2u,
           W_msg_u2i, W_root_u2i, b_u2i,
           W_msg_i2u, W_root_i2u, b_i2u):
    table = jnp.concatenate([x_user, x_item], axis=0)
    src_all = jnp.stack([
        edge_index_u2i[0].astype(jnp.int32),
        edge_index_i2u[0].astype(jnp.int32) + _N_USER,
    ])
    dst_all = jnp.stack([
        edge_index_u2i[1].astype(jnp.int32),
        edge_index_i2u[1].astype(jnp.int32),
    ])
    zeros = jnp.zeros((_N_ITEM, _D), jnp.float32)
    agg = _sc_aggregate(table, src_all, dst_all, zeros)
    out_item, out_user = _tc_epilogue(
        agg, x_item, x_user,
        W_msg_u2i, W_root_u2i, b_u2i,
        W_msg_i2u, W_root_i2u, b_i2u)
    return (out_user, out_item)


# speedup vs baseline: 4.6414x; 4.6414x over previous
import functools

import jax
import jax.numpy as jnp
from jax import lax
from jax.experimental import pallas as pl
from jax.experimental.pallas import tpu as pltpu
from jax.experimental.pallas import tpu_sc as plsc

_N_USER = 10000
_N_ITEM = 10000
_D = 128
_E = 320000

_NUM_TILES = 16
_EDGES_PER_TILE = _E // _NUM_TILES
_CHUNK = 80
_NCHUNKS = _EDGES_PER_TILE // _CHUNK
_N_PAD = 10240
_ROWS_PER_TILE = _N_PAD // _NUM_TILES


def _sc_aggregate(table, src_all, dst_all, zeros):
    n = zeros.shape[0]
    mesh = plsc.VectorSubcoreMesh(core_axis_name="c", subcore_axis_name="s")

    @functools.partial(
        pl.kernel,
        out_type=jax.ShapeDtypeStruct((2, n, _D), jnp.float32),
        mesh=mesh,
        scratch_types=[
            pltpu.VMEM((_CHUNK,), jnp.int32),
            pltpu.VMEM((_CHUNK,), jnp.int32),
            pltpu.VMEM((_CHUNK, _D), jnp.float32),
            pltpu.VMEM_SHARED((n, _D), jnp.float32),
            pltpu.SemaphoreType.DMA,
        ],
    )
    def agg_kernel(table_hbm, src_hbm, dst_hbm, zeros_hbm, out_hbm,
                   src_v, dst_v, rows_v, acc_sh, sem):
        c = lax.axis_index("c")
        s = lax.axis_index("s")
        rbase = s * _ROWS_PER_TILE
        pltpu.sync_copy(zeros_hbm.at[pl.ds(rbase, _ROWS_PER_TILE)],
                        acc_sh.at[pl.ds(rbase, _ROWS_PER_TILE)])
        plsc.subcore_barrier()

        ebase = c * _E + s * _EDGES_PER_TILE

        @pl.loop(0, _NCHUNKS)
        def _(i):
            e0 = ebase + i * _CHUNK
            pltpu.sync_copy(src_hbm.at[pl.ds(e0, _CHUNK)], src_v)
            pltpu.sync_copy(dst_hbm.at[pl.ds(e0, _CHUNK)], dst_v)
            pltpu.async_copy(table_hbm.at[src_v], rows_v, sem).wait()
            pltpu.sync_copy(rows_v, acc_sh.at[dst_v], add=True)

        plsc.subcore_barrier()
        pltpu.sync_copy(acc_sh.at[pl.ds(rbase, _ROWS_PER_TILE)],
                        out_hbm.at[c, pl.ds(rbase, _ROWS_PER_TILE)])

    return agg_kernel(table, src_all, dst_all, zeros)


def _affine_kernel(agg0_ref, agg1_ref, xi_ref, xu_ref,
                   wm0_ref, wr0_ref, b0_ref, wm1_ref, wr1_ref, b1_ref,
                   oi_ref, ou_ref):
    oi_ref[...] = (
        jnp.dot(agg0_ref[0], wm0_ref[...], preferred_element_type=jnp.float32)
        + jnp.dot(xi_ref[...], wr0_ref[...], preferred_element_type=jnp.float32)
        + b0_ref[...]
    )
    ou_ref[...] = (
        jnp.dot(agg1_ref[0], wm1_ref[...], preferred_element_type=jnp.float32)
        + jnp.dot(xu_ref[...], wr1_ref[...], preferred_element_type=jnp.float32)
        + b1_ref[...]
    )


def _tc_epilogue(agg, x_item, x_user, wm0, wr0, b0, wm1, wr1, b1):
    n = x_item.shape[0]
    blk = 2000
    grid = (n // blk,)
    row_spec = pl.BlockSpec((blk, _D), lambda i: (i, 0))
    w_spec = pl.BlockSpec((_D, _D), lambda i: (0, 0))
    b_spec = pl.BlockSpec((1, _D), lambda i: (0, 0))
    return pl.pallas_call(
        _affine_kernel,
        grid=grid,
        in_specs=[
            pl.BlockSpec((1, blk, _D), lambda i: (0, i, 0)),
            pl.BlockSpec((1, blk, _D), lambda i: (1, i, 0)),
            row_spec, row_spec,
            w_spec, w_spec, b_spec,
            w_spec, w_spec, b_spec,
        ],
        out_specs=[row_spec, row_spec],
        out_shape=[
            jax.ShapeDtypeStruct((n, _D), jnp.float32),
            jax.ShapeDtypeStruct((n, _D), jnp.float32),
        ],
    )(agg, agg, x_item, x_user, wm0, wr0, b0.reshape(1, _D),
      wm1, wr1, b1.reshape(1, _D))


def kernel(x_user, x_item, edge_index_u2i, edge_index_i2u,
           W_msg_u2i, W_root_u2i, b_u2i,
           W_msg_i2u, W_root_i2u, b_i2u):
    table = jnp.concatenate([x_user, x_item], axis=0)
    src_all = jnp.concatenate([
        edge_index_u2i[0].astype(jnp.int32),
        edge_index_i2u[0].astype(jnp.int32) + _N_USER,
    ])
    dst_all = jnp.concatenate([
        edge_index_u2i[1].astype(jnp.int32),
        edge_index_i2u[1].astype(jnp.int32),
    ])
    zeros = jnp.zeros((_N_PAD, _D), jnp.float32)
    agg = _sc_aggregate(table, src_all, dst_all, zeros)
    out_item, out_user = _tc_epilogue(
        agg, x_item, x_user,
        W_msg_u2i, W_root_u2i, b_u2i,
        W_msg_i2u, W_root_i2u, b_i2u)
    return (out_user, out_item)
